# baseline (speedup 1.0000x reference)
import functools

import jax
import jax.numpy as jnp
from jax import lax
from jax.experimental import pallas as pl
from jax.experimental.pallas import tpu as pltpu
from jax.experimental.pallas import tpu_sc as plsc

_VOCAB = 1000
_D = 128
_CTX = 200
_H1 = 256
_H2 = 128
_K = _CTX * _D
_RB = 128
_NSTEP = _H1 // _RB

_ROWS_PER_WORKER = 8
_NWORKERS = _CTX // _ROWS_PER_WORKER


@functools.lru_cache(maxsize=1)
def _make_sc_gather():
    mesh = plsc.VectorSubcoreMesh(core_axis_name="c", subcore_axis_name="s")

    @functools.partial(
        pl.kernel,
        mesh=mesh,
        out_type=jax.ShapeDtypeStruct((_CTX, _D), jnp.float32),
        scratch_types=[
            pltpu.VMEM((_ROWS_PER_WORKER,), jnp.int32),
            pltpu.VMEM((_ROWS_PER_WORKER, _D), jnp.float32),
            pltpu.SemaphoreType.DMA,
        ],
    )
    def _sc_gather(idx_hbm, table_hbm, out_hbm, idx_v, rows_v, sem):
        nc = 2
        wid = lax.axis_index("s") * nc + lax.axis_index("c")

        @pl.when(wid < _NWORKERS)
        def _():
            base = pl.multiple_of(wid * _ROWS_PER_WORKER, 8)
            pltpu.sync_copy(idx_hbm.at[pl.ds(base, _ROWS_PER_WORKER)], idx_v)
            pltpu.async_copy(table_hbm.at[idx_v], rows_v, sem).wait()
            pltpu.sync_copy(rows_v, out_hbm.at[pl.ds(base, _ROWS_PER_WORKER)])

    return _sc_gather


def _mlp_body(idx_ref, emb_ref, w1_ref, w2_ref, w3_ref, b1_ref, b2_ref, b3_ref,
              out_ref, x_ref, acc_ref):
    i = pl.program_id(0)

    @pl.when(i == 0)
    def _():
        for j in range(_CTX):
            r = idx_ref[j]
            x_ref[j:j + 1, :] = emb_ref[pl.ds(r, 1), :]

    local = jnp.zeros((_RB, _D), jnp.float32)
    for j in range(_CTX):
        sl = pl.ds(j * _D, _D)
        local = local + w1_ref[:, sl] * x_ref[j:j + 1, :]
    acc_ref[pl.ds(i * _RB, _RB), :] = local

    @pl.when(i == _NSTEP - 1)
    def _():
        acc_t = jnp.swapaxes(acc_ref[...], 0, 1)
        h1 = jnp.sum(acc_t, axis=0, keepdims=True) + b1_ref[...].reshape(1, _H1)
        h1 = jnp.maximum(h1, 0.0)
        h2 = lax.dot_general(h1, w2_ref[...], (((1,), (1,)), ((), ())),
                             preferred_element_type=jnp.float32)
        h2 = jnp.maximum(h2 + b2_ref[...].reshape(1, _H2), 0.0)
        logits = lax.dot_general(h2, w3_ref[...], (((1,), (1,)), ((), ())),
                                 preferred_element_type=jnp.float32)
        logits = logits + b3_ref[...].reshape(1, _VOCAB)
        m = jnp.max(logits, axis=1, keepdims=True)
        ssum = jnp.sum(jnp.exp(logits - m), axis=1, keepdims=True)
        out_ref[...] = logits - m - jnp.log(ssum)


def _mlp(idx, emb, W1, W2, W3, b1, b2, b3, interpret=False):
    return pl.pallas_call(
        _mlp_body,
        grid=(_NSTEP,),
        in_specs=[
            pl.BlockSpec(memory_space=pltpu.SMEM),
            pl.BlockSpec((_VOCAB, _D), lambda i: (0, 0)),
            pl.BlockSpec((_RB, _K), lambda i: (i, 0)),
            pl.BlockSpec((_H2, _H1), lambda i: (0, 0)),
            pl.BlockSpec((_VOCAB, _H2), lambda i: (0, 0)),
            pl.BlockSpec((_H1,), lambda i: (0,)),
            pl.BlockSpec((_H2,), lambda i: (0,)),
            pl.BlockSpec((_VOCAB,), lambda i: (0,)),
        ],
        out_specs=pl.BlockSpec((1, _VOCAB), lambda i: (0, 0)),
        out_shape=jax.ShapeDtypeStruct((1, _VOCAB), jnp.float32),
        scratch_shapes=[
            pltpu.VMEM((_CTX, _D), jnp.float32),
            pltpu.VMEM((_H1, _D), jnp.float32),
        ],
        interpret=interpret,
    )(idx, emb, W1, W2, W3, b1, b2, b3)


def kernel(inputs, emb, W1, b1, W2, b2, W3, b3):
    return _mlp(inputs, emb, W1, W2, W3, b1, b2, b3)


# speedup vs baseline: 1.0390x; 1.0390x over previous
import functools

import jax
import jax.numpy as jnp
from jax import lax
from jax.experimental import pallas as pl
from jax.experimental.pallas import tpu as pltpu
from jax.experimental.pallas import tpu_sc as plsc

_VOCAB = 1000
_D = 128
_CTX = 200
_H1 = 256
_H2 = 128
_K = _CTX * _D
_RB = 64
_NSTEP = _H1 // _RB

_ROWS_PER_WORKER = 8
_NWORKERS = _CTX // _ROWS_PER_WORKER


@functools.lru_cache(maxsize=1)
def _make_sc_gather():
    mesh = plsc.VectorSubcoreMesh(core_axis_name="c", subcore_axis_name="s")

    @functools.partial(
        pl.kernel,
        mesh=mesh,
        out_type=jax.ShapeDtypeStruct((_CTX, _D), jnp.float32),
        scratch_types=[
            pltpu.VMEM((_ROWS_PER_WORKER,), jnp.int32),
            pltpu.VMEM((_ROWS_PER_WORKER, _D), jnp.float32),
            pltpu.SemaphoreType.DMA,
        ],
    )
    def _sc_gather(idx_hbm, table_hbm, out_hbm, idx_v, rows_v, sem):
        nc = 2
        wid = lax.axis_index("s") * nc + lax.axis_index("c")

        @pl.when(wid < _NWORKERS)
        def _():
            base = pl.multiple_of(wid * _ROWS_PER_WORKER, 8)
            pltpu.sync_copy(idx_hbm.at[pl.ds(base, _ROWS_PER_WORKER)], idx_v)
            pltpu.async_copy(table_hbm.at[idx_v], rows_v, sem).wait()
            pltpu.sync_copy(rows_v, out_hbm.at[pl.ds(base, _ROWS_PER_WORKER)])

    return _sc_gather


def _mlp_body(idx_ref, emb_ref, w1_ref, w2_ref, w3_ref, b1_ref, b2_ref, b3_ref,
              out_ref, x_ref, acc_ref):
    i = pl.program_id(0)

    @pl.when(i == 0)
    def _():
        for j in range(_CTX):
            r = idx_ref[j]
            x_ref[j:j + 1, :] = emb_ref[pl.ds(r, 1), :]

    local = jnp.zeros((_RB, _D), jnp.float32)
    for j in range(_CTX):
        sl = pl.ds(j * _D, _D)
        local = local + w1_ref[:, sl] * x_ref[j:j + 1, :]
    acc_ref[pl.ds(i * _RB, _RB), :] = local

    @pl.when(i == _NSTEP - 1)
    def _():
        acc_t = jnp.swapaxes(acc_ref[...], 0, 1)
        h1 = jnp.sum(acc_t, axis=0, keepdims=True) + b1_ref[...].reshape(1, _H1)
        h1 = jnp.maximum(h1, 0.0)
        h2 = lax.dot_general(h1, w2_ref[...], (((1,), (1,)), ((), ())),
                             preferred_element_type=jnp.float32)
        h2 = jnp.maximum(h2 + b2_ref[...].reshape(1, _H2), 0.0)
        logits = lax.dot_general(h2, w3_ref[...], (((1,), (1,)), ((), ())),
                                 preferred_element_type=jnp.float32)
        logits = logits + b3_ref[...].reshape(1, _VOCAB)
        m = jnp.max(logits, axis=1, keepdims=True)
        ssum = jnp.sum(jnp.exp(logits - m), axis=1, keepdims=True)
        out_ref[...] = logits - m - jnp.log(ssum)


def _mlp(idx, emb, W1, W2, W3, b1, b2, b3, interpret=False):
    return pl.pallas_call(
        _mlp_body,
        grid=(_NSTEP,),
        in_specs=[
            pl.BlockSpec(memory_space=pltpu.SMEM),
            pl.BlockSpec((_VOCAB, _D), lambda i: (0, 0)),
            pl.BlockSpec((_RB, _K), lambda i: (i, 0)),
            pl.BlockSpec((_H2, _H1), lambda i: (0, 0)),
            pl.BlockSpec((_VOCAB, _H2), lambda i: (0, 0)),
            pl.BlockSpec((_H1,), lambda i: (0,)),
            pl.BlockSpec((_H2,), lambda i: (0,)),
            pl.BlockSpec((_VOCAB,), lambda i: (0,)),
        ],
        out_specs=pl.BlockSpec((1, _VOCAB), lambda i: (0, 0)),
        out_shape=jax.ShapeDtypeStruct((1, _VOCAB), jnp.float32),
        scratch_shapes=[
            pltpu.VMEM((_CTX, _D), jnp.float32),
            pltpu.VMEM((_H1, _D), jnp.float32),
        ],
        interpret=interpret,
    )(idx, emb, W1, W2, W3, b1, b2, b3)


def kernel(inputs, emb, W1, b1, W2, b2, W3, b3):
    return _mlp(inputs, emb, W1, W2, W3, b1, b2, b3)
